# baseline (speedup 1.0000x reference)
import functools

import jax
import jax.numpy as jnp
from jax import lax
from jax.experimental import pallas as pl
from jax.experimental.pallas import tpu as pltpu
from jax.experimental.pallas import tpu_sc as plsc

KSEL = 512
F32 = jnp.float32
BF16 = jnp.bfloat16
NEG = -1e30


def _mm(a, b):
    return jax.lax.dot_general(
        a.astype(BF16), b.astype(BF16),
        (((a.ndim - 1,), (0,)), ((), ())),
        preferred_element_type=F32)


def _sortable(g):
    bits = jax.lax.bitcast_convert_type(g, jnp.int32)
    return jnp.where(bits < 0, ~bits ^ jnp.int32(-2147483648), bits)


def _sc_gather(embed, idx):
    N = idx.shape[0]
    V, H = embed.shape
    NW = 32
    CH = 64
    rows_per_w = N // NW
    n_ch = rows_per_w // CH
    mesh = plsc.VectorSubcoreMesh(core_axis_name="c", subcore_axis_name="s")

    @functools.partial(
        pl.kernel, mesh=mesh,
        out_type=jax.ShapeDtypeStruct((N, H), F32),
        scratch_types=[
            pltpu.VMEM((CH,), jnp.int32),
            pltpu.VMEM((CH,), jnp.int32),
            pltpu.VMEM((CH, H), F32),
            pltpu.VMEM((CH, H), F32),
            pltpu.SemaphoreType.DMA,
            pltpu.SemaphoreType.DMA,
            pltpu.SemaphoreType.DMA,
            pltpu.SemaphoreType.DMA,
        ],
    )
    def k(idx_hbm, table_hbm, out_hbm, idx0, idx1, buf0, buf1,
          g0, g1, w0, w1):
        wid = lax.axis_index("s") * 2 + lax.axis_index("c")
        base = wid * rows_per_w

        def st(i):
            return pl.multiple_of(base + i * CH, CH)

        pltpu.sync_copy(idx_hbm.at[pl.ds(st(0), CH)], idx0)
        pltpu.async_copy(table_hbm.at[idx0], buf0, g0)

        def body(j, carry):
            c0 = 2 * j
            c1 = 2 * j + 1
            pltpu.sync_copy(idx_hbm.at[pl.ds(st(c1), CH)], idx1)

            @pl.when(j > 0)
            def _():
                pltpu.make_async_copy(buf1, out_hbm.at[pl.ds(st(c1 - 2), CH)],
                                      w1).wait()

            pltpu.make_async_copy(table_hbm.at[idx0], buf0, g0).wait()
            pltpu.async_copy(table_hbm.at[idx1], buf1, g1)
            pltpu.async_copy(buf0, out_hbm.at[pl.ds(st(c0), CH)], w0)

            @pl.when(j + 1 < n_ch // 2)
            def _():
                pltpu.sync_copy(idx_hbm.at[pl.ds(st(c0 + 2), CH)], idx0)

            pltpu.make_async_copy(buf0, out_hbm.at[pl.ds(st(c0), CH)],
                                  w0).wait()
            pltpu.make_async_copy(table_hbm.at[idx1], buf1, g1).wait()
            pltpu.async_copy(buf1, out_hbm.at[pl.ds(st(c1), CH)], w1)

            @pl.when(j + 1 < n_ch // 2)
            def _():
                pltpu.async_copy(table_hbm.at[idx0], buf0, g0)

            return carry

        lax.fori_loop(0, n_ch // 2, body, 0)
        pltpu.make_async_copy(buf1, out_hbm.at[pl.ds(st(n_ch - 1), CH)],
                              w1).wait()

    return k(idx, embed)


def _ffn_kernel(h0_ref, W1_ref, b1_ref, W2_ref, b2_ref, g_ref, be_ref,
                Wg_ref, bg_ref, h_ref, gate_ref):
    h0 = h0_ref[...]
    a = _mm(h0, W1_ref[...]) + b1_ref[...]
    a = jnp.maximum(a, 0.0)
    ff = _mm(a, W2_ref[...]) + b2_ref[...]
    x = h0 + ff
    mu = jnp.mean(x, axis=-1, keepdims=True)
    var = jnp.mean((x - mu) ** 2, axis=-1, keepdims=True)
    h = (x - mu) / jnp.sqrt(var + 1e-5) * g_ref[...] + be_ref[...]
    h_ref[...] = h.astype(BF16)
    gate_ref[0] = jax.lax.dot_general(
        Wg_ref[...].astype(BF16), h.astype(BF16),
        (((1,), (1,)), ((), ())),
        preferred_element_type=F32) + bg_ref[...]


def _ffn_chunk(h0_c, W1, b1, W2, b2, gamma, beta, Wg, bg,
               N, c, prev=None):
    NC, H = h0_c.shape
    BLK = 1024
    nb = NC // BLK
    off = c * nb
    const = lambda shape: pl.BlockSpec(shape, lambda i: tuple(0 for _ in shape))
    kern = _ffn_kernel if prev is None else (
        lambda hp, gp, *a: _ffn_kernel(*a))
    in_specs = [
        pl.BlockSpec((BLK, H), lambda i: (i, 0)),
        const((H, 2 * H)), const((2 * H,)),
        const((2 * H, H)), const((H,)),
        const((H,)), const((H,)),
        const((1, H)), const((1,)),
    ]
    args = (h0_c, W1, b1, W2, b2, gamma, beta, Wg, bg)
    aliases = {}
    if prev is not None:
        in_specs = [pl.BlockSpec(memory_space=pl.ANY),
                    pl.BlockSpec(memory_space=pl.ANY)] + in_specs
        args = prev + args
        aliases = {0: 0, 1: 1}
    return pl.pallas_call(
        kern,
        grid=(nb,),
        in_specs=in_specs,
        out_specs=[
            pl.BlockSpec((BLK, H), lambda i: (off + i, 0)),
            pl.BlockSpec((1, 1, BLK), lambda i: (off + i, 0, 0)),
        ],
        out_shape=[
            jax.ShapeDtypeStruct((N, H), BF16),
            jax.ShapeDtypeStruct((N // BLK, 1, BLK), F32),
        ],
        input_output_aliases=aliases,
    )(*args)


def _select_kernel(gate_ref, hlast_ref, Wq_ref, bq_ref,
                   q_ref, thr_ref, icut_ref):
    g = gate_ref[...]
    B, T = g.shape
    keys = _sortable(g)

    K0 = jnp.full((B, 1), -2147483648, jnp.int32)

    def bit_step(i, K):
        cand = K + (jnp.int32(1) << (31 - i))
        cnt = jnp.sum((keys >= cand).astype(jnp.int32), axis=1, keepdims=True)
        return jnp.where(cnt >= KSEL, cand, K)

    K = lax.fori_loop(0, 32, bit_step, K0, unroll=True)

    n_gt = jnp.sum((keys > K).astype(jnp.int32), axis=1, keepdims=True)
    need = KSEL - n_gt
    idx = lax.broadcasted_iota(jnp.int32, (B, T), 1)
    eq = keys == K

    I0 = jnp.zeros((B, 1), jnp.int32)

    def ibit_step(i, I):
        cand = I + (jnp.int32(1) << (13 - i))
        cnt = jnp.sum((eq & (idx < cand)).astype(jnp.int32),
                      axis=1, keepdims=True)
        return jnp.where(cnt <= need, cand, I)

    I = lax.fori_loop(0, 14, ibit_step, I0, unroll=True)

    q_ref[...] = _mm(hlast_ref[...], Wq_ref[...]) + bq_ref[...]
    thr_ref[...] = jnp.broadcast_to(K, thr_ref.shape)
    icut_ref[...] = jnp.broadcast_to(I, icut_ref.shape)


def _select(gate, hlast, Wq, bq):
    B, T = gate.shape
    H = hlast.shape[1]
    return pl.pallas_call(
        _select_kernel,
        out_shape=[
            jax.ShapeDtypeStruct((B, H), F32),
            jax.ShapeDtypeStruct((B, 128), jnp.int32),
            jax.ShapeDtypeStruct((B, 128), jnp.int32),
        ],
    )(gate, hlast, Wq, bq)


def _ctx_kernel(h_ref, gate_ref, q_ref, thr_ref, icut_ref, ctx_ref,
                m_sc, z_sc, acc_sc):
    i = pl.program_id(0)
    nb = pl.num_programs(0)
    B, BLKT, H = h_ref.shape

    @pl.when(i == 0)
    def _init():
        m_sc[...] = jnp.full_like(m_sc, NEG)
        z_sc[...] = jnp.zeros_like(z_sc)
        acc_sc[...] = jnp.zeros_like(acc_sc)

    for b in range(B):
        h = h_ref[b]
        g = gate_ref[b, 0]
        qrow = q_ref[b]
        K = thr_ref[b][:, :1]
        I = icut_ref[b][:, :1]

        keys = _sortable(g)
        tidx = i * BLKT + lax.broadcasted_iota(jnp.int32, (1, BLKT), 1)
        sel = (keys > K) | ((keys == K) & (tidx < I))

        s = jax.lax.dot_general(
            qrow.astype(BF16), h.astype(BF16),
            (((1,), (1,)), ((), ())),
            preferred_element_type=F32)

        m_old = m_sc[b:b + 1, :1]
        s_m = jnp.where(sel, s, NEG)
        m_new = jnp.maximum(m_old, jnp.max(s_m, axis=1, keepdims=True))
        scale = jnp.exp(m_old - m_new)
        p = jnp.where(sel, jnp.exp(s - m_new), 0.0)
        z_new = z_sc[b:b + 1, :1] * scale + jnp.sum(p, axis=1, keepdims=True)
        pa = jax.lax.dot_general(
            p.astype(BF16), h.astype(BF16),
            (((1,), (0,)), ((), ())),
            preferred_element_type=F32)
        acc_sc[b:b + 1, :] = acc_sc[b:b + 1, :] * scale + pa
        m_sc[b:b + 1, :] = jnp.broadcast_to(m_new, (1, 128))
        z_sc[b:b + 1, :] = jnp.broadcast_to(z_new, (1, 128))

    @pl.when(i == nb - 1)
    def _fin():
        ctx_ref[...] = acc_sc[...] / z_sc[...][:, :1]


def _ctx(h3, gate4, qc, thr, icut):
    B, T, H = h3.shape
    BLKT = 1024
    nt = T // BLKT
    return pl.pallas_call(
        _ctx_kernel,
        grid=(nt,),
        in_specs=[
            pl.BlockSpec((B, BLKT, H), lambda i: (0, i, 0)),
            pl.BlockSpec((B, 1, 1, BLKT), lambda i: (0, i, 0, 0)),
            pl.BlockSpec((B, 1, H), lambda i: (0, 0, 0)),
            pl.BlockSpec((B, 1, 128), lambda i: (0, 0, 0)),
            pl.BlockSpec((B, 1, 128), lambda i: (0, 0, 0)),
        ],
        out_specs=pl.BlockSpec((B, H), lambda i: (0, 0)),
        out_shape=jax.ShapeDtypeStruct((B, H), F32),
        scratch_shapes=[
            pltpu.VMEM((B, 128), F32),
            pltpu.VMEM((B, 128), F32),
            pltpu.VMEM((B, H), F32),
        ],
    )(h3, gate4, qc, thr, icut)


def _proj_kernel(ctx_ref, WoT_ref, bo_ref, out_ref):
    out_ref[...] = jax.lax.dot_general(
        ctx_ref[...].astype(BF16), WoT_ref[...].astype(BF16),
        (((1,), (1,)), ((), ())),
        preferred_element_type=F32) + bo_ref[...]


def _proj(ctx, Wo, bo):
    B, H = ctx.shape
    V = Wo.shape[1]
    BLKV = 8192
    return pl.pallas_call(
        _proj_kernel,
        grid=(pl.cdiv(V, BLKV),),
        in_specs=[
            pl.BlockSpec((B, H), lambda i: (0, 0)),
            pl.BlockSpec((BLKV, H), lambda i: (i, 0)),
            pl.BlockSpec((1, BLKV), lambda i: (0, i)),
        ],
        out_specs=pl.BlockSpec((B, BLKV), lambda i: (0, i)),
        out_shape=jax.ShapeDtypeStruct((B, V), F32),
    )(ctx, Wo.T, bo.reshape(1, V))


def kernel(seq, embed, W1, b1, W2, b2, gamma, beta, Wg, bg, Wq, bq, Wo, bo):
    B, T = seq.shape
    V, H = embed.shape
    N = B * T
    idx = seq.reshape(N).astype(jnp.int32)
    NCHUNK = 4
    NC = N // NCHUNK
    W1b, W2b = W1.astype(BF16), W2.astype(BF16)
    Wgb = Wg.astype(BF16).reshape(1, H)
    prev = None
    for c in range(NCHUNK):
        h0_c = _sc_gather(embed, lax.slice(idx, (c * NC,), ((c + 1) * NC,)))
        prev = _ffn_chunk(h0_c, W1b, b1, W2b, b2, gamma, beta, Wgb, bg,
                          N, c, prev)
    h, gate = prev
    gate2 = gate.reshape(B, T)
    h3 = h.reshape(B, T, H)
    hlast = h3[:, -1, :]
    q, thr, icut = _select(gate2, hlast, Wq, bq)
    ctx = _ctx(h3, gate.reshape(B, T // 1024, 1, 1024), q.reshape(B, 1, H),
               thr.reshape(B, 1, 128), icut.reshape(B, 1, 128))
    return _proj(ctx, Wo, bo)


# speedup vs baseline: 1.0146x; 1.0146x over previous
import functools

import jax
import jax.numpy as jnp
from jax import lax
from jax.experimental import pallas as pl
from jax.experimental.pallas import tpu as pltpu
from jax.experimental.pallas import tpu_sc as plsc

KSEL = 512
F32 = jnp.float32
BF16 = jnp.bfloat16
NEG = -1e30


def _mm(a, b):
    return jax.lax.dot_general(
        a.astype(BF16), b.astype(BF16),
        (((a.ndim - 1,), (0,)), ((), ())),
        preferred_element_type=F32)


def _sortable(g):
    bits = jax.lax.bitcast_convert_type(g, jnp.int32)
    return jnp.where(bits < 0, ~bits ^ jnp.int32(-2147483648), bits)


def _sc_gather(embed, idx):
    N = idx.shape[0]
    V, H = embed.shape
    NW = 32
    CH = 64
    rows_per_w = N // NW
    n_ch = rows_per_w // CH
    mesh = plsc.VectorSubcoreMesh(core_axis_name="c", subcore_axis_name="s")

    @functools.partial(
        pl.kernel, mesh=mesh,
        out_type=jax.ShapeDtypeStruct((N, H), F32),
        scratch_types=[
            pltpu.VMEM((CH,), jnp.int32),
            pltpu.VMEM((CH,), jnp.int32),
            pltpu.VMEM((CH, H), F32),
            pltpu.VMEM((CH, H), F32),
            pltpu.SemaphoreType.DMA,
            pltpu.SemaphoreType.DMA,
            pltpu.SemaphoreType.DMA,
            pltpu.SemaphoreType.DMA,
        ],
    )
    def k(idx_hbm, table_hbm, out_hbm, idx0, idx1, buf0, buf1,
          g0, g1, w0, w1):
        wid = lax.axis_index("s") * 2 + lax.axis_index("c")
        base = wid * rows_per_w

        def st(i):
            return pl.multiple_of(base + i * CH, CH)

        pltpu.sync_copy(idx_hbm.at[pl.ds(st(0), CH)], idx0)
        pltpu.async_copy(table_hbm.at[idx0], buf0, g0)

        def body(j, carry):
            c0 = 2 * j
            c1 = 2 * j + 1
            pltpu.sync_copy(idx_hbm.at[pl.ds(st(c1), CH)], idx1)

            @pl.when(j > 0)
            def _():
                pltpu.make_async_copy(buf1, out_hbm.at[pl.ds(st(c1 - 2), CH)],
                                      w1).wait()

            pltpu.make_async_copy(table_hbm.at[idx0], buf0, g0).wait()
            pltpu.async_copy(table_hbm.at[idx1], buf1, g1)
            pltpu.async_copy(buf0, out_hbm.at[pl.ds(st(c0), CH)], w0)

            @pl.when(j + 1 < n_ch // 2)
            def _():
                pltpu.sync_copy(idx_hbm.at[pl.ds(st(c0 + 2), CH)], idx0)

            pltpu.make_async_copy(buf0, out_hbm.at[pl.ds(st(c0), CH)],
                                  w0).wait()
            pltpu.make_async_copy(table_hbm.at[idx1], buf1, g1).wait()
            pltpu.async_copy(buf1, out_hbm.at[pl.ds(st(c1), CH)], w1)

            @pl.when(j + 1 < n_ch // 2)
            def _():
                pltpu.async_copy(table_hbm.at[idx0], buf0, g0)

            return carry

        lax.fori_loop(0, n_ch // 2, body, 0)
        pltpu.make_async_copy(buf1, out_hbm.at[pl.ds(st(n_ch - 1), CH)],
                              w1).wait()

    return k(idx, embed)


def _ffn_kernel(h0_ref, W1_ref, b1_ref, W2_ref, b2_ref, g_ref, be_ref,
                Wg_ref, bg_ref, h_ref, gate_ref):
    h0 = h0_ref[...]
    a = _mm(h0, W1_ref[...]) + b1_ref[...]
    a = jnp.maximum(a, 0.0)
    ff = _mm(a, W2_ref[...]) + b2_ref[...]
    x = h0 + ff
    mu = jnp.mean(x, axis=-1, keepdims=True)
    var = jnp.mean((x - mu) ** 2, axis=-1, keepdims=True)
    h = (x - mu) / jnp.sqrt(var + 1e-5) * g_ref[...] + be_ref[...]
    h_ref[...] = h.astype(BF16)
    gate_ref[0] = jax.lax.dot_general(
        Wg_ref[...].astype(BF16), h.astype(BF16),
        (((1,), (1,)), ((), ())),
        preferred_element_type=F32) + bg_ref[...]


def _ffn_chunk(h0_c, W1, b1, W2, b2, gamma, beta, Wg, bg,
               N, c, prev=None):
    NC, H = h0_c.shape
    BLK = 1024
    nb = NC // BLK
    off = c * nb
    const = lambda shape: pl.BlockSpec(shape, lambda i: tuple(0 for _ in shape))
    kern = _ffn_kernel if prev is None else (
        lambda hp, gp, *a: _ffn_kernel(*a))
    in_specs = [
        pl.BlockSpec((BLK, H), lambda i: (i, 0)),
        const((H, 2 * H)), const((2 * H,)),
        const((2 * H, H)), const((H,)),
        const((H,)), const((H,)),
        const((1, H)), const((1,)),
    ]
    args = (h0_c, W1, b1, W2, b2, gamma, beta, Wg, bg)
    aliases = {}
    if prev is not None:
        in_specs = [pl.BlockSpec(memory_space=pl.ANY),
                    pl.BlockSpec(memory_space=pl.ANY)] + in_specs
        args = prev + args
        aliases = {0: 0, 1: 1}
    return pl.pallas_call(
        kern,
        grid=(nb,),
        in_specs=in_specs,
        out_specs=[
            pl.BlockSpec((BLK, H), lambda i: (off + i, 0)),
            pl.BlockSpec((1, 1, BLK), lambda i: (off + i, 0, 0)),
        ],
        out_shape=[
            jax.ShapeDtypeStruct((N, H), BF16),
            jax.ShapeDtypeStruct((N // BLK, 1, BLK), F32),
        ],
        input_output_aliases=aliases,
    )(*args)


def _select_kernel(gate_ref, hlast_ref, Wq_ref, bq_ref,
                   q_ref, thr_ref, icut_ref):
    g = gate_ref[...]
    B, T = g.shape
    keys = _sortable(g)

    K0 = jnp.full((B, 1), -2147483648, jnp.int32)

    def bit_step(i, K):
        cand = K + (jnp.int32(1) << (31 - i))
        cnt = jnp.sum((keys >= cand).astype(jnp.int32), axis=1, keepdims=True)
        return jnp.where(cnt >= KSEL, cand, K)

    K = lax.fori_loop(0, 32, bit_step, K0, unroll=True)

    n_gt = jnp.sum((keys > K).astype(jnp.int32), axis=1, keepdims=True)
    need = KSEL - n_gt
    idx = lax.broadcasted_iota(jnp.int32, (B, T), 1)
    eq = keys == K

    I0 = jnp.zeros((B, 1), jnp.int32)

    def ibit_step(i, I):
        cand = I + (jnp.int32(1) << (13 - i))
        cnt = jnp.sum((eq & (idx < cand)).astype(jnp.int32),
                      axis=1, keepdims=True)
        return jnp.where(cnt <= need, cand, I)

    I = lax.fori_loop(0, 14, ibit_step, I0, unroll=True)

    q_ref[...] = _mm(hlast_ref[...], Wq_ref[...]) + bq_ref[...]
    thr_ref[...] = jnp.broadcast_to(K, thr_ref.shape)
    icut_ref[...] = jnp.broadcast_to(I, icut_ref.shape)


def _select(gate, hlast, Wq, bq):
    B, T = gate.shape
    H = hlast.shape[1]
    return pl.pallas_call(
        _select_kernel,
        out_shape=[
            jax.ShapeDtypeStruct((B, H), F32),
            jax.ShapeDtypeStruct((B, 128), jnp.int32),
            jax.ShapeDtypeStruct((B, 128), jnp.int32),
        ],
    )(gate, hlast, Wq, bq)


def _ctx_kernel(h_ref, gate_ref, q_ref, thr_ref, icut_ref, ctx_ref,
                m_sc, z_sc, acc_sc):
    i = pl.program_id(0)
    nb = pl.num_programs(0)
    B, BLKT, H = h_ref.shape

    @pl.when(i == 0)
    def _init():
        m_sc[...] = jnp.full_like(m_sc, NEG)
        z_sc[...] = jnp.zeros_like(z_sc)
        acc_sc[...] = jnp.zeros_like(acc_sc)

    for b in range(B):
        h = h_ref[b]
        g = gate_ref[b, 0]
        qrow = q_ref[b]
        K = thr_ref[b][:, :1]
        I = icut_ref[b][:, :1]

        keys = _sortable(g)
        tidx = i * BLKT + lax.broadcasted_iota(jnp.int32, (1, BLKT), 1)
        sel = (keys > K) | ((keys == K) & (tidx < I))

        s = jax.lax.dot_general(
            qrow.astype(BF16), h.astype(BF16),
            (((1,), (1,)), ((), ())),
            preferred_element_type=F32)

        m_old = m_sc[b:b + 1, :1]
        s_m = jnp.where(sel, s, NEG)
        m_new = jnp.maximum(m_old, jnp.max(s_m, axis=1, keepdims=True))
        scale = jnp.exp(m_old - m_new)
        p = jnp.where(sel, jnp.exp(s - m_new), 0.0)
        z_new = z_sc[b:b + 1, :1] * scale + jnp.sum(p, axis=1, keepdims=True)
        pa = jax.lax.dot_general(
            p.astype(BF16), h.astype(BF16),
            (((1,), (0,)), ((), ())),
            preferred_element_type=F32)
        acc_sc[b:b + 1, :] = acc_sc[b:b + 1, :] * scale + pa
        m_sc[b:b + 1, :] = jnp.broadcast_to(m_new, (1, 128))
        z_sc[b:b + 1, :] = jnp.broadcast_to(z_new, (1, 128))

    @pl.when(i == nb - 1)
    def _fin():
        ctx_ref[...] = acc_sc[...] / z_sc[...][:, :1]


def _ctx(h3, gate4, qc, thr, icut):
    B, T, H = h3.shape
    BLKT = 1024
    nt = T // BLKT
    return pl.pallas_call(
        _ctx_kernel,
        grid=(nt,),
        in_specs=[
            pl.BlockSpec((B, BLKT, H), lambda i: (0, i, 0)),
            pl.BlockSpec((B, 1, 1, BLKT), lambda i: (0, i, 0, 0)),
            pl.BlockSpec((B, 1, H), lambda i: (0, 0, 0)),
            pl.BlockSpec((B, 1, 128), lambda i: (0, 0, 0)),
            pl.BlockSpec((B, 1, 128), lambda i: (0, 0, 0)),
        ],
        out_specs=pl.BlockSpec((B, H), lambda i: (0, 0)),
        out_shape=jax.ShapeDtypeStruct((B, H), F32),
        scratch_shapes=[
            pltpu.VMEM((B, 128), F32),
            pltpu.VMEM((B, 128), F32),
            pltpu.VMEM((B, H), F32),
        ],
    )(h3, gate4, qc, thr, icut)


def _proj_kernel(ctx_ref, WoT_ref, bo_ref, out_ref):
    out_ref[...] = jax.lax.dot_general(
        ctx_ref[...].astype(BF16), WoT_ref[...].astype(BF16),
        (((1,), (1,)), ((), ())),
        preferred_element_type=F32) + bo_ref[...]


def _proj(ctx, Wo, bo):
    B, H = ctx.shape
    V = Wo.shape[1]
    BLKV = 4096
    return pl.pallas_call(
        _proj_kernel,
        grid=(pl.cdiv(V, BLKV),),
        in_specs=[
            pl.BlockSpec((B, H), lambda i: (0, 0)),
            pl.BlockSpec((BLKV, H), lambda i: (i, 0)),
            pl.BlockSpec((1, BLKV), lambda i: (0, i)),
        ],
        out_specs=pl.BlockSpec((B, BLKV), lambda i: (0, i)),
        out_shape=jax.ShapeDtypeStruct((B, V), F32),
    )(ctx, Wo.T, bo.reshape(1, V))


def kernel(seq, embed, W1, b1, W2, b2, gamma, beta, Wg, bg, Wq, bq, Wo, bo):
    B, T = seq.shape
    V, H = embed.shape
    N = B * T
    idx = seq.reshape(N).astype(jnp.int32)
    NCHUNK = 4
    NC = N // NCHUNK
    W1b, W2b = W1.astype(BF16), W2.astype(BF16)
    Wgb = Wg.astype(BF16).reshape(1, H)
    prev = None
    for c in range(NCHUNK):
        h0_c = _sc_gather(embed, lax.slice(idx, (c * NC,), ((c + 1) * NC,)))
        prev = _ffn_chunk(h0_c, W1b, b1, W2b, b2, gamma, beta, Wgb, bg,
                          N, c, prev)
    h, gate = prev
    gate2 = gate.reshape(B, T)
    h3 = h.reshape(B, T, H)
    hlast = h3[:, -1, :]
    q, thr, icut = _select(gate2, hlast, Wq, bq)
    ctx = _ctx(h3, gate.reshape(B, T // 1024, 1, 1024), q.reshape(B, 1, H),
               thr.reshape(B, 1, 128), icut.reshape(B, 1, 128))
    return _proj(ctx, Wo, bo)
